# baseline (speedup 1.0000x reference)
import functools

import jax
import jax.numpy as jnp
from jax import lax
from jax.experimental import pallas as pl
from jax.experimental.pallas import tpu as pltpu
from jax.experimental.pallas import tpu_sc as plsc

_C0_FRAC = 0.84
_LANES = 16
_TILES = 16
_CORES = 2
_CHUNK = 128


def _z_matmul(x, w_all, w_root, bias2):
    n, in_ch = x.shape
    k = w_all.shape[1]
    out_ch = w_root.shape[1]
    bn = 2000
    assert n % bn == 0

    def body(x_ref, w_ref, wr_ref, b_ref, z_ref, r_ref):
        xb = x_ref[...]
        z_ref[...] = jnp.dot(xb, w_ref[...],
                             preferred_element_type=jnp.float32)
        r_ref[...] = jnp.dot(xb, wr_ref[...],
                             preferred_element_type=jnp.float32) + b_ref[...]

    return pl.pallas_call(
        body,
        grid=(n // bn,),
        in_specs=[
            pl.BlockSpec((bn, in_ch), lambda i: (i, 0)),
            pl.BlockSpec((in_ch, k), lambda i: (0, 0)),
            pl.BlockSpec((in_ch, out_ch), lambda i: (0, 0)),
            pl.BlockSpec((1, out_ch), lambda i: (0, 0)),
        ],
        out_specs=[
            pl.BlockSpec((bn, k), lambda i: (i, 0)),
            pl.BlockSpec((bn, out_ch), lambda i: (i, 0)),
        ],
        out_shape=[
            jax.ShapeDtypeStruct((n, k), jnp.float32),
            jax.ShapeDtypeStruct((n, out_ch), jnp.float32),
        ],
    )(x, w_all, w_root, bias2)


@functools.lru_cache(maxsize=None)
def _build_sc_agg(n_pad, out_ch, nrow_tile, cpt0, cpt1):
    mesh = plsc.VectorSubcoreMesh(core_axis_name="c", subcore_axis_name="s")
    ncopy = nrow_tile // _CHUNK
    vpr = out_ch // _LANES

    @functools.partial(
        pl.kernel,
        mesh=mesh,
        out_type=(
            jax.ShapeDtypeStruct((_CORES, n_pad, out_ch), jnp.float32),
            jax.ShapeDtypeStruct((_CORES, n_pad), jnp.float32),
        ),
        scratch_types=[
            pltpu.VMEM((_CHUNK, out_ch), jnp.float32),
            pltpu.VMEM((_CHUNK, out_ch), jnp.float32),
            pltpu.VMEM((_CHUNK,), jnp.int32),
            pltpu.VMEM((_CHUNK,), jnp.int32),
            pltpu.VMEM((_CHUNK,), jnp.int32),
            pltpu.VMEM((_CHUNK,), jnp.int32),
            pltpu.VMEM((_CHUNK,), jnp.float32),
            pltpu.VMEM((nrow_tile,), jnp.float32),
            pltpu.VMEM_SHARED((n_pad, out_ch), jnp.float32),
            pltpu.VMEM_SHARED((n_pad,), jnp.float32),
            pltpu.SemaphoreType.DMA,
            pltpu.SemaphoreType.DMA,
        ],
    )
    def sc_agg(z_hbm, gidx_hbm, dst_hbm, acc_hbm, deg_hbm,
               rows0_v, rows1_v, gidx0_v, gidx1_v, dst0_v, dst1_v,
               ones_v, dz_v, acc_sh, deg_sh, sem0, sem1):
        c = lax.axis_index("c")
        s = lax.axis_index("s")
        w = c * _TILES + s
        zeros16 = jnp.zeros((_LANES,), jnp.float32)
        ones16 = jnp.ones((_LANES,), jnp.float32)
        def _zero_row(i, carry):
            for j in range(vpr):
                rows0_v[i, pl.ds(j * _LANES, _LANES)] = zeros16
            return carry

        lax.fori_loop(0, _CHUNK, _zero_row, 0)
        for j in range(_CHUNK // _LANES):
            ones_v[pl.ds(j * _LANES, _LANES)] = ones16

        def _zero_d(i, carry):
            dz_v[pl.ds(i * _LANES, _LANES)] = zeros16
            return carry

        lax.fori_loop(0, nrow_tile // _LANES, _zero_d, 0)

        row0 = s * nrow_tile
        for j in range(ncopy):
            pltpu.sync_copy(rows0_v, acc_sh.at[pl.ds(row0 + j * _CHUNK, _CHUNK)])
        pltpu.sync_copy(dz_v, deg_sh.at[pl.ds(row0, nrow_tile)])

        my_cpt = jnp.where(c == 0, cpt0, cpt1)
        ebase = jnp.where(c == 0, s * cpt0,
                          _TILES * cpt0 + s * cpt1) * _CHUNK
        sets = ((rows0_v, gidx0_v, dst0_v, sem0),
                (rows1_v, gidx1_v, dst1_v, sem1))

        pltpu.sync_copy(gidx_hbm.at[pl.ds(ebase, _CHUNK)], gidx0_v)
        pltpu.sync_copy(dst_hbm.at[pl.ds(ebase, _CHUNK)], dst0_v)
        pltpu.async_copy(z_hbm.at[gidx0_v], rows0_v, sem0)
        plsc.subcore_barrier()

        def _pair(i, carry):
            for k in range(2):
                g = i * 2 + k
                rows_v, gidx_v, dst_v, sem = sets[k]
                rows_n, gidx_n, dst_n, sem_n = sets[1 - k]
                off = ebase + (g + 1) * _CHUNK
                pltpu.sync_copy(gidx_hbm.at[pl.ds(off, _CHUNK)], gidx_n)
                pltpu.sync_copy(dst_hbm.at[pl.ds(off, _CHUNK)], dst_n)
                pltpu.async_copy(z_hbm.at[gidx_n], rows_n, sem_n)
                pltpu.make_async_copy(z_hbm.at[gidx_v], rows_v, sem).wait()
                pltpu.sync_copy(rows_v, acc_sh.at[dst_v], add=True)
                pltpu.sync_copy(ones_v, deg_sh.at[dst_v], add=True)
            return carry

        lax.fori_loop(0, my_cpt // 2, _pair, 0)
        pltpu.make_async_copy(z_hbm.at[gidx0_v], rows0_v, sem0).wait()
        plsc.subcore_barrier()

        for j in range(ncopy):
            r = row0 + j * _CHUNK
            pltpu.sync_copy(acc_sh.at[pl.ds(r, _CHUNK)],
                            acc_hbm.at[c, pl.ds(r, _CHUNK)])
        pltpu.sync_copy(deg_sh.at[pl.ds(row0, nrow_tile)],
                        deg_hbm.at[c, pl.ds(row0, nrow_tile)])

    return sc_agg


def _finalize(acc, deg_t, root):
    ncores, n, out_ch = acc.shape
    bn = 2000
    assert n % bn == 0

    def body(a_ref, d_ref, r_ref, o_ref):
        a = a_ref[0] + a_ref[1]
        d = d_ref[...]
        dsum = d[:, 0:1] + d[:, 1:2]
        inv = 1.0 / jnp.maximum(dsum, 1.0)
        o_ref[...] = a * inv + r_ref[...]

    return pl.pallas_call(
        body,
        grid=(n // bn,),
        in_specs=[
            pl.BlockSpec((ncores, bn, out_ch), lambda i: (0, i, 0)),
            pl.BlockSpec((bn, ncores), lambda i: (i, 0)),
            pl.BlockSpec((bn, out_ch), lambda i: (i, 0)),
        ],
        out_specs=pl.BlockSpec((bn, out_ch), lambda i: (i, 0)),
        out_shape=jax.ShapeDtypeStruct((n, out_ch), jnp.float32),
    )(acc, deg_t, root)


def kernel(x, edge_index, num_nodes, edge_type, weight, comp, bias):
    n, in_ch = x.shape
    out_ch = weight.shape[-1]
    nrel, nbases = comp.shape
    e = edge_index.shape[1]

    src = edge_index[0]
    dst = jnp.minimum(edge_index[1], num_nodes - 1).astype(jnp.int32)
    gidx = (src * nrel + edge_type).astype(jnp.int32)

    w_rel = jnp.einsum("rb,bio->rio", comp, weight[:nbases])
    w_all = jnp.transpose(w_rel, (1, 0, 2)).reshape(in_ch, nrel * out_ch)

    z, root = _z_matmul(x, w_all, weight[-1], bias.reshape(1, out_ch))
    z = z.reshape(n * nrel, out_ch)

    nrow_tile = -(-(n + 1) // (_TILES * _CHUNK)) * _CHUNK
    n_pad = nrow_tile * _TILES
    pair_chunks = -(-e // (_TILES * _CHUNK))
    pair_chunks += pair_chunks % 2
    cpt0 = 2 * max(1, round(pair_chunks * _C0_FRAC / 2))
    cpt1 = pair_chunks - cpt0
    e_pad = _TILES * pair_chunks * _CHUNK
    pad = e_pad + _CHUNK - e
    gidx = jnp.concatenate([gidx, jnp.zeros((pad,), jnp.int32)])
    pad_dst = n + jnp.arange(pad, dtype=jnp.int32) % (n_pad - n)
    dst = jnp.concatenate([dst, pad_dst])

    acc, deg = _build_sc_agg(n_pad, out_ch, nrow_tile, cpt0, cpt1)(
        z, gidx, dst)

    return _finalize(acc[:, :n], deg[:, :n].T, root)


# speedup vs baseline: 1.0352x; 1.0352x over previous
import functools

import jax
import jax.numpy as jnp
from jax import lax
from jax.experimental import pallas as pl
from jax.experimental.pallas import tpu as pltpu
from jax.experimental.pallas import tpu_sc as plsc

_C0_FRAC = 0.78
_LANES = 16
_TILES = 16
_CORES = 2
_CHUNK = 128


def _z_matmul(x, w_all, w_root, bias2):
    n, in_ch = x.shape
    k = w_all.shape[1]
    out_ch = w_root.shape[1]
    bn = 2000
    assert n % bn == 0

    def body(x_ref, w_ref, wr_ref, b_ref, z_ref, r_ref):
        xb = x_ref[...]
        z_ref[...] = jnp.dot(xb, w_ref[...],
                             preferred_element_type=jnp.float32)
        r_ref[...] = jnp.dot(xb, wr_ref[...],
                             preferred_element_type=jnp.float32) + b_ref[...]

    return pl.pallas_call(
        body,
        grid=(n // bn,),
        in_specs=[
            pl.BlockSpec((bn, in_ch), lambda i: (i, 0)),
            pl.BlockSpec((in_ch, k), lambda i: (0, 0)),
            pl.BlockSpec((in_ch, out_ch), lambda i: (0, 0)),
            pl.BlockSpec((1, out_ch), lambda i: (0, 0)),
        ],
        out_specs=[
            pl.BlockSpec((bn, k), lambda i: (i, 0)),
            pl.BlockSpec((bn, out_ch), lambda i: (i, 0)),
        ],
        out_shape=[
            jax.ShapeDtypeStruct((n, k), jnp.float32),
            jax.ShapeDtypeStruct((n, out_ch), jnp.float32),
        ],
    )(x, w_all, w_root, bias2)


@functools.lru_cache(maxsize=None)
def _build_sc_agg(n_pad, out_ch, nrow_tile, cpt0, cpt1):
    mesh = plsc.VectorSubcoreMesh(core_axis_name="c", subcore_axis_name="s")
    ncopy = nrow_tile // _CHUNK
    vpr = out_ch // _LANES

    @functools.partial(
        pl.kernel,
        mesh=mesh,
        out_type=(
            jax.ShapeDtypeStruct((_CORES, n_pad, out_ch), jnp.float32),
            jax.ShapeDtypeStruct((_CORES, n_pad), jnp.float32),
        ),
        scratch_types=[
            pltpu.VMEM((_CHUNK, out_ch), jnp.float32),
            pltpu.VMEM((_CHUNK, out_ch), jnp.float32),
            pltpu.VMEM((_CHUNK,), jnp.int32),
            pltpu.VMEM((_CHUNK,), jnp.int32),
            pltpu.VMEM((_CHUNK,), jnp.int32),
            pltpu.VMEM((_CHUNK,), jnp.int32),
            pltpu.VMEM((_CHUNK,), jnp.float32),
            pltpu.VMEM((nrow_tile,), jnp.float32),
            pltpu.VMEM_SHARED((n_pad, out_ch), jnp.float32),
            pltpu.VMEM_SHARED((n_pad,), jnp.float32),
            pltpu.SemaphoreType.DMA,
            pltpu.SemaphoreType.DMA,
        ],
    )
    def sc_agg(z_hbm, gidx_hbm, dst_hbm, acc_hbm, deg_hbm,
               rows0_v, rows1_v, gidx0_v, gidx1_v, dst0_v, dst1_v,
               ones_v, dz_v, acc_sh, deg_sh, sem0, sem1):
        c = lax.axis_index("c")
        s = lax.axis_index("s")
        w = c * _TILES + s
        zeros16 = jnp.zeros((_LANES,), jnp.float32)
        ones16 = jnp.ones((_LANES,), jnp.float32)
        def _zero_row(i, carry):
            for j in range(vpr):
                rows0_v[i, pl.ds(j * _LANES, _LANES)] = zeros16
            return carry

        lax.fori_loop(0, _CHUNK, _zero_row, 0)
        for j in range(_CHUNK // _LANES):
            ones_v[pl.ds(j * _LANES, _LANES)] = ones16

        def _zero_d(i, carry):
            dz_v[pl.ds(i * _LANES, _LANES)] = zeros16
            return carry

        lax.fori_loop(0, nrow_tile // _LANES, _zero_d, 0)

        row0 = s * nrow_tile
        for j in range(ncopy):
            pltpu.sync_copy(rows0_v, acc_sh.at[pl.ds(row0 + j * _CHUNK, _CHUNK)])
        pltpu.sync_copy(dz_v, deg_sh.at[pl.ds(row0, nrow_tile)])

        my_cpt = jnp.where(c == 0, cpt0, cpt1)
        ebase = jnp.where(c == 0, s * cpt0,
                          _TILES * cpt0 + s * cpt1) * _CHUNK
        sets = ((rows0_v, gidx0_v, dst0_v, sem0),
                (rows1_v, gidx1_v, dst1_v, sem1))

        pltpu.sync_copy(gidx_hbm.at[pl.ds(ebase, _CHUNK)], gidx0_v)
        pltpu.sync_copy(dst_hbm.at[pl.ds(ebase, _CHUNK)], dst0_v)
        pltpu.async_copy(z_hbm.at[gidx0_v], rows0_v, sem0)
        plsc.subcore_barrier()

        def _pair(i, carry):
            for k in range(2):
                g = i * 2 + k
                rows_v, gidx_v, dst_v, sem = sets[k]
                rows_n, gidx_n, dst_n, sem_n = sets[1 - k]
                off = ebase + (g + 1) * _CHUNK
                pltpu.sync_copy(gidx_hbm.at[pl.ds(off, _CHUNK)], gidx_n)
                pltpu.sync_copy(dst_hbm.at[pl.ds(off, _CHUNK)], dst_n)
                pltpu.async_copy(z_hbm.at[gidx_n], rows_n, sem_n)
                pltpu.make_async_copy(z_hbm.at[gidx_v], rows_v, sem).wait()
                pltpu.sync_copy(rows_v, acc_sh.at[dst_v], add=True)
                pltpu.sync_copy(ones_v, deg_sh.at[dst_v], add=True)
            return carry

        lax.fori_loop(0, my_cpt // 2, _pair, 0)
        pltpu.make_async_copy(z_hbm.at[gidx0_v], rows0_v, sem0).wait()
        plsc.subcore_barrier()

        for j in range(ncopy):
            r = row0 + j * _CHUNK
            pltpu.sync_copy(acc_sh.at[pl.ds(r, _CHUNK)],
                            acc_hbm.at[c, pl.ds(r, _CHUNK)])
        pltpu.sync_copy(deg_sh.at[pl.ds(row0, nrow_tile)],
                        deg_hbm.at[c, pl.ds(row0, nrow_tile)])

    return sc_agg


def _finalize(acc, deg_t, root):
    ncores, n, out_ch = acc.shape
    bn = 2000
    assert n % bn == 0

    def body(a_ref, d_ref, r_ref, o_ref):
        a = a_ref[0] + a_ref[1]
        d = d_ref[...]
        dsum = d[:, 0:1] + d[:, 1:2]
        inv = 1.0 / jnp.maximum(dsum, 1.0)
        o_ref[...] = a * inv + r_ref[...]

    return pl.pallas_call(
        body,
        grid=(n // bn,),
        in_specs=[
            pl.BlockSpec((ncores, bn, out_ch), lambda i: (0, i, 0)),
            pl.BlockSpec((bn, ncores), lambda i: (i, 0)),
            pl.BlockSpec((bn, out_ch), lambda i: (i, 0)),
        ],
        out_specs=pl.BlockSpec((bn, out_ch), lambda i: (i, 0)),
        out_shape=jax.ShapeDtypeStruct((n, out_ch), jnp.float32),
    )(acc, deg_t, root)


def kernel(x, edge_index, num_nodes, edge_type, weight, comp, bias):
    n, in_ch = x.shape
    out_ch = weight.shape[-1]
    nrel, nbases = comp.shape
    e = edge_index.shape[1]

    src = edge_index[0]
    dst = jnp.minimum(edge_index[1], num_nodes - 1).astype(jnp.int32)
    gidx = (src * nrel + edge_type).astype(jnp.int32)

    w_rel = jnp.einsum("rb,bio->rio", comp, weight[:nbases])
    w_all = jnp.transpose(w_rel, (1, 0, 2)).reshape(in_ch, nrel * out_ch)

    z, root = _z_matmul(x, w_all, weight[-1], bias.reshape(1, out_ch))
    z = z.reshape(n * nrel, out_ch)

    nrow_tile = -(-(n + 1) // (_TILES * _CHUNK)) * _CHUNK
    n_pad = nrow_tile * _TILES
    pair_chunks = -(-e // (_TILES * _CHUNK))
    pair_chunks += pair_chunks % 2
    cpt0 = 2 * max(1, round(pair_chunks * _C0_FRAC / 2))
    cpt1 = pair_chunks - cpt0
    e_pad = _TILES * pair_chunks * _CHUNK
    pad = e_pad + _CHUNK - e
    gidx = jnp.concatenate([gidx, jnp.zeros((pad,), jnp.int32)])
    pad_dst = n + jnp.arange(pad, dtype=jnp.int32) % (n_pad - n)
    dst = jnp.concatenate([dst, pad_dst])

    acc, deg = _build_sc_agg(n_pad, out_ch, nrow_tile, cpt0, cpt1)(
        z, gidx, dst)

    return _finalize(acc[:, :n], deg[:, :n].T, root)
